# baseline (speedup 1.0000x reference)
import jax
import jax.numpy as jnp
from jax.experimental import pallas as pl
from jax.experimental.pallas import tpu as pltpu
from jax.experimental.pallas import tpu_sc as plsc

B, N = 8, 8192
G, K = 512, 32
GB = 16
NGBLK = G // GB
NC = N // 128

DP = 8
NIDX = B * G * K
SC_NC, SC_NS = 2, 16
NW = SC_NC * SC_NS
ROWS_W = NIDX // NW


def _fps_kernel(xyzT_ref, cent_ref, dists_ref):
    x = xyzT_ref[0]
    y = xyzT_ref[1]
    z = xyzT_ref[2]
    dists_ref[...] = jnp.full((B, N), 1e10, jnp.float32)
    iota = jax.lax.broadcasted_iota(jnp.int32, (B, N), 1)

    def body(i, c):
        cx, cy, cz = c
        cent_ref[i] = jnp.concatenate([cx, cy, cz], axis=-1)
        d = (x - cx) ** 2 + (y - cy) ** 2 + (z - cz) ** 2
        dists = jnp.minimum(dists_ref[...], d)
        dists_ref[...] = dists
        m = jnp.max(dists, axis=-1, keepdims=True)
        j = jnp.min(jnp.where(dists == m, iota, N), axis=-1, keepdims=True)
        sel = iota == j
        ncx = jnp.sum(jnp.where(sel, x, 0.0), -1, keepdims=True)
        ncy = jnp.sum(jnp.where(sel, y, 0.0), -1, keepdims=True)
        ncz = jnp.sum(jnp.where(sel, z, 0.0), -1, keepdims=True)
        return (ncx, ncy, ncz)

    c0 = (x[:, 0:1], y[:, 0:1], z[:, 0:1])
    jax.lax.fori_loop(0, G, body, c0, unroll=False)


def _knn_kernel(xyzB_ref, cent_ref, out_ref, dd_ref):
    x = xyzB_ref[0, 0:1, :]
    y = xyzB_ref[0, 1:2, :]
    z = xyzB_ref[0, 2:3, :]
    c = cent_ref[0]
    cx = c[:, 0:1]
    cy = c[:, 1:2]
    cz = c[:, 2:3]
    xn2 = (x * x + y * y) + z * z
    cn2 = (cx * cx + cy * cy) + cz * cz
    dot = jax.lax.dot_general(
        c, xyzB_ref[0], (((1,), (0,)), ((), ())),
        preferred_element_type=jnp.float32)
    d0 = (cn2 + xn2) - 2.0 * dot

    dd_ref[...] = d0.reshape(GB * NC, 128)
    cm = jnp.min(d0.reshape(GB, NC, 128), axis=-1)
    i_nc = jax.lax.broadcasted_iota(jnp.int32, (GB, NC), 1)
    i128 = jax.lax.broadcasted_iota(jnp.int32, (GB, 128), 1)
    i_flat = jax.lax.broadcasted_iota(jnp.int32, (GB, GB * NC), 1)
    rowbase = jax.lax.broadcasted_iota(jnp.int32, (GB, 1), 0) * NC

    hist = []
    for t in range(K):
        mc = jnp.min(cm, -1, keepdims=True)
        jc = jnp.min(jnp.where(cm == mc, i_nc, NC), -1, keepdims=True)
        oh = (i_flat == rowbase + jc).astype(jnp.float32)
        chunk = jax.lax.dot_general(
            oh, dd_ref[...], (((1,), (0,)), ((), ())),
            precision=jax.lax.Precision.HIGHEST,
            preferred_element_type=jnp.float32)
        glane = jc * 128 + i128
        for h in hist:
            chunk = jnp.where(h == glane, jnp.inf, chunk)
        m2 = jnp.min(chunk, -1, keepdims=True)
        j = jnp.min(jnp.where(chunk == m2, i128, 128), -1, keepdims=True)
        g = jc * 128 + j
        out_ref[0, 0, t] = g
        hist.append(g)
        ncm = jnp.min(jnp.where(i128 == j, jnp.inf, chunk), -1, keepdims=True)
        cm = jnp.where(i_nc == jc, ncm, cm)


def _gather_kernel(x_hbm, y_hbm, z_hbm, idx_hbm, ox_hbm, oy_hbm, oz_hbm,
                   idx_v, vx, vy, vz, sem):
    wid = jax.lax.axis_index("s") * SC_NC + jax.lax.axis_index("c")
    base = wid * ROWS_W
    pltpu.sync_copy(idx_hbm.at[pl.ds(base, ROWS_W)], idx_v)
    pltpu.async_copy(x_hbm.at[idx_v], vx, sem).wait()
    pltpu.async_copy(y_hbm.at[idx_v], vy, sem).wait()
    pltpu.async_copy(z_hbm.at[idx_v], vz, sem).wait()
    pltpu.sync_copy(vx, ox_hbm.at[pl.ds(base, ROWS_W)])
    pltpu.sync_copy(vy, oy_hbm.at[pl.ds(base, ROWS_W)])
    pltpu.sync_copy(vz, oz_hbm.at[pl.ds(base, ROWS_W)])


def _norm_kernel(nb_ref, cent_ref, ab_ref, out_ref):
    cx = cent_ref[0, :, 0:1]
    cy = cent_ref[0, :, 1:2]
    cz = cent_ref[0, :, 2:3]
    dx = nb_ref[0, 0] - cx
    dy = nb_ref[0, 1] - cy
    dz = nb_ref[0, 2] - cz
    cnt = 3.0 * G * K
    mean = (jnp.sum(dx) + jnp.sum(dy) + jnp.sum(dz)) / cnt
    var = (jnp.sum((dx - mean) ** 2) + jnp.sum((dy - mean) ** 2)
           + jnp.sum((dz - mean) ** 2)) / (cnt - 1.0)
    denom = jnp.sqrt(var) + 1e-05
    out_ref[0, 0] = (dx / denom) * ab_ref[0] + ab_ref[3]
    out_ref[0, 1] = (dy / denom) * ab_ref[1] + ab_ref[4]
    out_ref[0, 2] = (dz / denom) * ab_ref[2] + ab_ref[5]
    out_ref[0, 3] = jnp.broadcast_to(cx, (G, K))
    out_ref[0, 4] = jnp.broadcast_to(cy, (G, K))
    out_ref[0, 5] = jnp.broadcast_to(cz, (G, K))


def kernel(xyz, alpha, beta):
    xyzT = jnp.transpose(xyz, (2, 0, 1))

    cent = pl.pallas_call(
        _fps_kernel,
        out_shape=jax.ShapeDtypeStruct((G, B, 3), jnp.float32),
        scratch_shapes=[pltpu.VMEM((B, N), jnp.float32)],
    )(xyzT)

    xyzB = jnp.transpose(xyz, (0, 2, 1))
    centB = jnp.transpose(cent, (1, 0, 2))

    idx = pl.pallas_call(
        _knn_kernel,
        grid=(B, NGBLK),
        in_specs=[
            pl.BlockSpec((1, 3, N), lambda b, g: (b, 0, 0)),
            pl.BlockSpec((1, GB, 3), lambda b, g: (b, g, 0)),
        ],
        out_specs=pl.BlockSpec((1, 1, K, GB, 1), lambda b, g: (b, g, 0, 0, 0)),
        out_shape=jax.ShapeDtypeStruct((B, NGBLK, K, GB, 1), jnp.int32),
        scratch_shapes=[
            pltpu.VMEM((GB * NC, 128), jnp.float32),
        ],
        compiler_params=pltpu.CompilerParams(
            dimension_semantics=("parallel", "parallel")),
    )(xyzB, centB)

    idxT = jnp.transpose(idx[..., 0], (0, 1, 3, 2)).reshape(B, G, K)
    idx_flat = (idxT + jnp.arange(B, dtype=jnp.int32)[:, None, None] * N
                ).reshape(NIDX)

    xpl = xyz[:, :, 0].reshape(B * N)
    ypl = xyz[:, :, 1].reshape(B * N)
    zpl = xyz[:, :, 2].reshape(B * N)

    gx, gy, gz = pl.kernel(
        _gather_kernel,
        mesh=plsc.VectorSubcoreMesh(core_axis_name="c", subcore_axis_name="s"),
        out_type=[
            jax.ShapeDtypeStruct((NIDX,), jnp.float32),
            jax.ShapeDtypeStruct((NIDX,), jnp.float32),
            jax.ShapeDtypeStruct((NIDX,), jnp.float32),
        ],
        scratch_types=[
            pltpu.VMEM((ROWS_W,), jnp.int32),
            pltpu.VMEM((ROWS_W,), jnp.float32),
            pltpu.VMEM((ROWS_W,), jnp.float32),
            pltpu.VMEM((ROWS_W,), jnp.float32),
            pltpu.SemaphoreType.DMA,
        ],
    )(xpl, ypl, zpl, idx_flat)

    nbT = jnp.stack([gx, gy, gz], axis=0).reshape(3, B, G, K)
    nbT = jnp.transpose(nbT, (1, 0, 2, 3))
    ab = jnp.concatenate([alpha.reshape(3), beta.reshape(3)])

    out = pl.pallas_call(
        _norm_kernel,
        grid=(B,),
        in_specs=[
            pl.BlockSpec((1, 3, G, K), lambda b: (b, 0, 0, 0)),
            pl.BlockSpec((1, G, 3), lambda b: (b, 0, 0)),
            pl.BlockSpec(memory_space=pltpu.SMEM),
        ],
        out_specs=pl.BlockSpec((1, 6, G, K), lambda b: (b, 0, 0, 0)),
        out_shape=jax.ShapeDtypeStruct((B, 6, G, K), jnp.float32),
        compiler_params=pltpu.CompilerParams(
            dimension_semantics=("parallel",)),
    )(nbT, centB, ab)

    neighborhood = jnp.transpose(out, (0, 2, 3, 1))
    center = jnp.transpose(cent, (1, 0, 2))
    return neighborhood, center


# speedup vs baseline: 1.4032x; 1.4032x over previous
import jax
import jax.numpy as jnp
from jax.experimental import pallas as pl
from jax.experimental.pallas import tpu as pltpu
from jax.experimental.pallas import tpu_sc as plsc

B, N = 8, 8192
G, K = 512, 32
GB = 8
NGBLK = G // GB

DP = 8
NIDX = B * G * K
SC_NC, SC_NS = 2, 16
NW = SC_NC * SC_NS
ROWS_W = NIDX // NW


def _fps_kernel(xyzT_ref, cent_ref, dists_ref):
    x = xyzT_ref[0]
    y = xyzT_ref[1]
    z = xyzT_ref[2]
    dists_ref[...] = jnp.full((B, N), 1e10, jnp.float32)
    iota = jax.lax.broadcasted_iota(jnp.int32, (B, N), 1)

    def body(i, c):
        cx, cy, cz = c
        cent_ref[i] = jnp.concatenate([cx, cy, cz], axis=-1)
        d = (x - cx) ** 2 + (y - cy) ** 2 + (z - cz) ** 2
        dists = jnp.minimum(dists_ref[...], d)
        dists_ref[...] = dists
        m = jnp.max(dists, axis=-1, keepdims=True)
        j = jnp.min(jnp.where(dists == m, iota, N), axis=-1, keepdims=True)
        sel = iota == j
        ncx = jnp.sum(jnp.where(sel, x, 0.0), -1, keepdims=True)
        ncy = jnp.sum(jnp.where(sel, y, 0.0), -1, keepdims=True)
        ncz = jnp.sum(jnp.where(sel, z, 0.0), -1, keepdims=True)
        return (ncx, ncy, ncz)

    c0 = (x[:, 0:1], y[:, 0:1], z[:, 0:1])
    jax.lax.fori_loop(0, G, body, c0, unroll=False)


def _knn_kernel(xyzB_ref, cent_ref, out_ref):
    x = xyzB_ref[0, 0:1, :]
    y = xyzB_ref[0, 1:2, :]
    z = xyzB_ref[0, 2:3, :]
    c = cent_ref[0]
    cx = c[:, 0:1]
    cy = c[:, 1:2]
    cz = c[:, 2:3]
    xn2 = (x * x + y * y) + z * z
    cn2 = (cx * cx + cy * cy) + cz * cz
    dot = jax.lax.dot_general(
        c, xyzB_ref[0], (((1,), (0,)), ((), ())),
        preferred_element_type=jnp.float32)
    d = (cn2 + xn2) - 2.0 * dot

    iota = jax.lax.broadcasted_iota(jnp.int32, (GB, N), 1)
    for t in range(K):
        m = jnp.min(d, -1, keepdims=True)
        j = jnp.min(jnp.where(d == m, iota, N), -1, keepdims=True)
        out_ref[0, 0, t] = j
        d = jnp.where(iota == j, jnp.inf, d)


def _gather_kernel(x_hbm, y_hbm, z_hbm, idx_hbm, ox_hbm, oy_hbm, oz_hbm,
                   idx_v, vx, vy, vz, sem):
    wid = jax.lax.axis_index("s") * SC_NC + jax.lax.axis_index("c")
    base = wid * ROWS_W
    pltpu.sync_copy(idx_hbm.at[pl.ds(base, ROWS_W)], idx_v)
    pltpu.async_copy(x_hbm.at[idx_v], vx, sem).wait()
    pltpu.async_copy(y_hbm.at[idx_v], vy, sem).wait()
    pltpu.async_copy(z_hbm.at[idx_v], vz, sem).wait()
    pltpu.sync_copy(vx, ox_hbm.at[pl.ds(base, ROWS_W)])
    pltpu.sync_copy(vy, oy_hbm.at[pl.ds(base, ROWS_W)])
    pltpu.sync_copy(vz, oz_hbm.at[pl.ds(base, ROWS_W)])


def _norm_kernel(nb_ref, cent_ref, ab_ref, out_ref):
    cx = cent_ref[0, :, 0:1]
    cy = cent_ref[0, :, 1:2]
    cz = cent_ref[0, :, 2:3]
    dx = nb_ref[0, 0] - cx
    dy = nb_ref[0, 1] - cy
    dz = nb_ref[0, 2] - cz
    cnt = 3.0 * G * K
    mean = (jnp.sum(dx) + jnp.sum(dy) + jnp.sum(dz)) / cnt
    var = (jnp.sum((dx - mean) ** 2) + jnp.sum((dy - mean) ** 2)
           + jnp.sum((dz - mean) ** 2)) / (cnt - 1.0)
    denom = jnp.sqrt(var) + 1e-05
    out_ref[0, 0] = (dx / denom) * ab_ref[0] + ab_ref[3]
    out_ref[0, 1] = (dy / denom) * ab_ref[1] + ab_ref[4]
    out_ref[0, 2] = (dz / denom) * ab_ref[2] + ab_ref[5]
    out_ref[0, 3] = jnp.broadcast_to(cx, (G, K))
    out_ref[0, 4] = jnp.broadcast_to(cy, (G, K))
    out_ref[0, 5] = jnp.broadcast_to(cz, (G, K))


def kernel(xyz, alpha, beta):
    xyzT = jnp.transpose(xyz, (2, 0, 1))

    cent = pl.pallas_call(
        _fps_kernel,
        out_shape=jax.ShapeDtypeStruct((G, B, 3), jnp.float32),
        scratch_shapes=[pltpu.VMEM((B, N), jnp.float32)],
    )(xyzT)

    xyzB = jnp.transpose(xyz, (0, 2, 1))
    centB = jnp.transpose(cent, (1, 0, 2))

    idx = pl.pallas_call(
        _knn_kernel,
        grid=(B, NGBLK),
        in_specs=[
            pl.BlockSpec((1, 3, N), lambda b, g: (b, 0, 0)),
            pl.BlockSpec((1, GB, 3), lambda b, g: (b, g, 0)),
        ],
        out_specs=pl.BlockSpec((1, 1, K, GB, 1), lambda b, g: (b, g, 0, 0, 0)),
        out_shape=jax.ShapeDtypeStruct((B, NGBLK, K, GB, 1), jnp.int32),
        compiler_params=pltpu.CompilerParams(
            dimension_semantics=("parallel", "parallel")),
    )(xyzB, centB)

    idxT = jnp.transpose(idx[..., 0], (0, 1, 3, 2)).reshape(B, G, K)
    idx_flat = (idxT + jnp.arange(B, dtype=jnp.int32)[:, None, None] * N
                ).reshape(NIDX)

    xpl = xyz[:, :, 0].reshape(B * N)
    ypl = xyz[:, :, 1].reshape(B * N)
    zpl = xyz[:, :, 2].reshape(B * N)

    gx, gy, gz = pl.kernel(
        _gather_kernel,
        mesh=plsc.VectorSubcoreMesh(core_axis_name="c", subcore_axis_name="s"),
        out_type=[
            jax.ShapeDtypeStruct((NIDX,), jnp.float32),
            jax.ShapeDtypeStruct((NIDX,), jnp.float32),
            jax.ShapeDtypeStruct((NIDX,), jnp.float32),
        ],
        scratch_types=[
            pltpu.VMEM((ROWS_W,), jnp.int32),
            pltpu.VMEM((ROWS_W,), jnp.float32),
            pltpu.VMEM((ROWS_W,), jnp.float32),
            pltpu.VMEM((ROWS_W,), jnp.float32),
            pltpu.SemaphoreType.DMA,
        ],
    )(xpl, ypl, zpl, idx_flat)

    nbT = jnp.stack([gx, gy, gz], axis=0).reshape(3, B, G, K)
    nbT = jnp.transpose(nbT, (1, 0, 2, 3))
    ab = jnp.concatenate([alpha.reshape(3), beta.reshape(3)])

    out = pl.pallas_call(
        _norm_kernel,
        grid=(B,),
        in_specs=[
            pl.BlockSpec((1, 3, G, K), lambda b: (b, 0, 0, 0)),
            pl.BlockSpec((1, G, 3), lambda b: (b, 0, 0)),
            pl.BlockSpec(memory_space=pltpu.SMEM),
        ],
        out_specs=pl.BlockSpec((1, 6, G, K), lambda b: (b, 0, 0, 0)),
        out_shape=jax.ShapeDtypeStruct((B, 6, G, K), jnp.float32),
        compiler_params=pltpu.CompilerParams(
            dimension_semantics=("parallel",)),
    )(nbT, centB, ab)

    neighborhood = jnp.transpose(out, (0, 2, 3, 1))
    center = jnp.transpose(cent, (1, 0, 2))
    return neighborhood, center
